# baseline (speedup 1.0000x reference)
import functools

import jax
import jax.numpy as jnp
from jax.experimental import pallas as pl
from jax.experimental.pallas import tpu as pltpu

_Q = 0.1
_CB = 512
_RC = 512

_TOPBIT = -2**31
_MAXI32 = 2**31 - 1


def _trim_kernel(n_ref, klo_ref, khi_ref, x_ref, o_ref, key_ref, hi_ref):
    b = pl.program_id(0)
    n = n_ref[b]
    klo = klo_ref[b]
    khi = khi_ref[b]

    S = x_ref.shape[1]
    CB = x_ref.shape[2]
    NCMAX = S // _RC
    nchunks = (n + (_RC - 1)) // _RC

    crow = jax.lax.broadcasted_iota(jnp.int32, (_RC, CB), 0)

    ones = jnp.ones((1, _RC), jnp.float32)
    ones_bf = jnp.ones((1, _RC), jnp.bfloat16)
    klo_f = klo.astype(jnp.float32)
    khi_f = khi.astype(jnp.float32)
    dot = functools.partial(
        jax.lax.dot_general,
        dimension_numbers=(((1,), (0,)), ((), ())),
        preferred_element_type=jnp.float32,
    )

    def run(nc):
        for ci in range(nc):
            x = x_ref[0, pl.ds(ci * _RC, _RC), :]
            i = jax.lax.bitcast_convert_type(x, jnp.int32)
            skey = jnp.where(i < 0, i ^ _MAXI32, i)
            skey = jnp.where(crow + ci * _RC < n, skey, _MAXI32)
            key_ref[pl.ds(ci * _RC, _RC), :] = skey
            hi_ref[pl.ds(ci * _RC, _RC), :] = (
                jax.lax.shift_right_arithmetic(skey, 16).astype(jnp.int16))

        one_bf = jnp.bfloat16(1.0)
        zero_bf = jnp.bfloat16(0.0)

        def step_hi(t, carry):
            p_lo, p_hi = carry
            bitv = jnp.left_shift(jnp.int32(1), 15 - t)
            t_lo = ((p_lo | bitv) - 32768).astype(jnp.int16)
            t_hi = ((p_hi | bitv) - 32768).astype(jnp.int16)
            c_lo = jnp.zeros((1, CB), jnp.float32)
            c_hi = c_lo
            for ci in range(nc):
                h = hi_ref[pl.ds(ci * _RC, _RC), :]
                c_lo = c_lo + dot(ones_bf, jnp.where(h < t_lo, one_bf, zero_bf))
                c_hi = c_hi + dot(ones_bf, jnp.where(h < t_hi, one_bf, zero_bf))
            p_lo = jnp.where(c_lo <= klo_f, p_lo | bitv, p_lo)
            p_hi = jnp.where(c_hi <= khi_f, p_hi | bitv, p_hi)
            return p_lo, p_hi

        zero = jnp.zeros((1, CB), jnp.int32)
        p_lo, p_hi = jax.lax.fori_loop(0, 16, step_hi, (zero, zero))
        p_lo = jnp.left_shift(p_lo, 16)
        p_hi = jnp.left_shift(p_hi, 16)

        def step(t, carry):
            p_lo, p_hi = carry
            bitv = jnp.left_shift(jnp.int32(1), 31 - t)
            t_lo = (p_lo | bitv) ^ _TOPBIT
            t_hi = (p_hi | bitv) ^ _TOPBIT
            c_lo = jnp.zeros((1, CB), jnp.float32)
            c_hi = c_lo
            for ci in range(nc):
                k = key_ref[pl.ds(ci * _RC, _RC), :]
                c_lo = c_lo + dot(ones, jnp.where(k < t_lo, 1.0, 0.0))
                c_hi = c_hi + dot(ones, jnp.where(k < t_hi, 1.0, 0.0))
            p_lo = jnp.where(c_lo <= klo_f, p_lo | bitv, p_lo)
            p_hi = jnp.where(c_hi <= khi_f, p_hi | bitv, p_hi)
            return p_lo, p_hi

        p_lo, p_hi = jax.lax.fori_loop(16, 32, step, (p_lo, p_hi))

        bits_lo = jnp.where(p_lo < 0, p_lo & _MAXI32, ~p_lo)
        bits_hi = jnp.where(p_hi < 0, p_hi & _MAXI32, ~p_hi)
        a_min = jax.lax.bitcast_convert_type(bits_lo, jnp.float32)
        a_max = jax.lax.bitcast_convert_type(bits_hi, jnp.float32)

        ssum = jnp.zeros((1, CB), jnp.float32)
        scnt = ssum
        for ci in range(nc):
            x = x_ref[0, pl.ds(ci * _RC, _RC), :]
            valid = crow + ci * _RC < n
            kept = valid & (x >= a_min) & (x <= a_max)
            ssum = ssum + jnp.sum(jnp.where(kept, x, 0.0), axis=0, keepdims=True)
            scnt = scnt + jnp.sum(jnp.where(kept, 1.0, 0.0), axis=0, keepdims=True)
        o_ref[0, 0, 0, :] = (ssum / scnt)[0]

    jax.lax.switch(nchunks - 1, [functools.partial(run, nc)
                                 for nc in range(1, NCMAX + 1)])


def _quantile_index(nf, q):
    qv = jnp.float32(q) * (nf - 1.0)
    low = jnp.floor(qv)
    high = jnp.ceil(qv)
    hw = qv - low
    low = jnp.clip(low, 0.0, nf - 1.0)
    high = jnp.clip(high, 0.0, nf - 1.0)
    return jnp.where(hw <= 0.5, low, high).astype(jnp.int32)


@jax.jit
def kernel(outputs, lens):
    B, S, D = outputs.shape
    length = lens * S
    n = jnp.where(length != 1, jnp.floor(length).astype(jnp.int32) + 1, S)
    nf = n.astype(jnp.float32)
    k_lo = _quantile_index(nf, _Q)
    k_hi = _quantile_index(nf, 1.0 - _Q)

    grid_spec = pltpu.PrefetchScalarGridSpec(
        num_scalar_prefetch=3,
        grid=(B, D // _CB),
        in_specs=[
            pl.BlockSpec((1, S, _CB), lambda b, c, *_: (b, 0, c)),
        ],
        out_specs=pl.BlockSpec((1, 1, 1, _CB), lambda b, c, *_: (b, 0, 0, c)),
        scratch_shapes=[pltpu.VMEM((S, _CB), jnp.int32),
                        pltpu.VMEM((S, _CB), jnp.int16)],
    )
    out = pl.pallas_call(
        _trim_kernel,
        grid_spec=grid_spec,
        out_shape=jax.ShapeDtypeStruct((B, 1, 1, D), jnp.float32),
        compiler_params=pltpu.CompilerParams(
            dimension_semantics=("parallel", "parallel"),
        ),
    )(n, k_lo, k_hi, outputs)
    return out


# speedup vs baseline: 1.0152x; 1.0152x over previous
import functools

import jax
import jax.numpy as jnp
from jax.experimental import pallas as pl
from jax.experimental.pallas import tpu as pltpu

_Q = 0.1
_CB = 512
_RC = 512

_TOPBIT = -2**31
_MAXI32 = 2**31 - 1


def _trim_kernel(n_ref, klo_ref, khi_ref, x_ref, o_ref, key_ref):
    b = pl.program_id(0)
    n = n_ref[b]
    klo = klo_ref[b]
    khi = khi_ref[b]

    S = x_ref.shape[1]
    CB = x_ref.shape[2]
    NCMAX = S // _RC
    nchunks = (n + (_RC - 1)) // _RC

    crow = jax.lax.broadcasted_iota(jnp.int32, (_RC, CB), 0)

    ones = jnp.ones((1, _RC), jnp.float32)
    klo_f = klo.astype(jnp.float32)
    khi_f = khi.astype(jnp.float32)
    dot = functools.partial(
        jax.lax.dot_general,
        dimension_numbers=(((1,), (0,)), ((), ())),
        preferred_element_type=jnp.float32,
    )

    def run(nc):
        for ci in range(nc):
            x = x_ref[0, pl.ds(ci * _RC, _RC), :]
            i = jax.lax.bitcast_convert_type(x, jnp.int32)
            skey = jnp.where(i < 0, i ^ _MAXI32, i)
            skey = jnp.where(crow + ci * _RC < n, skey, _MAXI32)
            key_ref[pl.ds(ci * _RC, _RC), :] = skey

        def step(t, carry):
            p_lo, p_hi = carry
            bitv = jnp.left_shift(jnp.int32(1), 31 - t)
            t_lo = (p_lo | bitv) ^ _TOPBIT
            t_hi = (p_hi | bitv) ^ _TOPBIT
            c_lo = jnp.zeros((1, CB), jnp.float32)
            c_hi = c_lo
            for ci in range(nc):
                k = key_ref[pl.ds(ci * _RC, _RC), :]
                c_lo = c_lo + dot(ones, jnp.where(k < t_lo, 1.0, 0.0))
                c_hi = c_hi + dot(ones, jnp.where(k < t_hi, 1.0, 0.0))
            p_lo = jnp.where(c_lo <= klo_f, p_lo | bitv, p_lo)
            p_hi = jnp.where(c_hi <= khi_f, p_hi | bitv, p_hi)
            return p_lo, p_hi

        zero = jnp.zeros((1, CB), jnp.int32)
        p_lo, p_hi = jax.lax.fori_loop(0, 32, step, (zero, zero))

        bits_lo = jnp.where(p_lo < 0, p_lo & _MAXI32, ~p_lo)
        bits_hi = jnp.where(p_hi < 0, p_hi & _MAXI32, ~p_hi)
        a_min = jax.lax.bitcast_convert_type(bits_lo, jnp.float32)
        a_max = jax.lax.bitcast_convert_type(bits_hi, jnp.float32)

        ssum = jnp.zeros((1, CB), jnp.float32)
        scnt = ssum
        for ci in range(nc):
            x = x_ref[0, pl.ds(ci * _RC, _RC), :]
            valid = crow + ci * _RC < n
            kept = valid & (x >= a_min) & (x <= a_max)
            ssum = ssum + jnp.sum(jnp.where(kept, x, 0.0), axis=0, keepdims=True)
            scnt = scnt + jnp.sum(jnp.where(kept, 1.0, 0.0), axis=0, keepdims=True)
        o_ref[0, 0, 0, :] = (ssum / scnt)[0]

    jax.lax.switch(nchunks - 1, [functools.partial(run, nc)
                                 for nc in range(1, NCMAX + 1)])


def _quantile_index(nf, q):
    qv = jnp.float32(q) * (nf - 1.0)
    low = jnp.floor(qv)
    high = jnp.ceil(qv)
    hw = qv - low
    low = jnp.clip(low, 0.0, nf - 1.0)
    high = jnp.clip(high, 0.0, nf - 1.0)
    return jnp.where(hw <= 0.5, low, high).astype(jnp.int32)


@jax.jit
def kernel(outputs, lens):
    B, S, D = outputs.shape
    length = lens * S
    n = jnp.where(length != 1, jnp.floor(length).astype(jnp.int32) + 1, S)
    nf = n.astype(jnp.float32)
    k_lo = _quantile_index(nf, _Q)
    k_hi = _quantile_index(nf, 1.0 - _Q)

    grid_spec = pltpu.PrefetchScalarGridSpec(
        num_scalar_prefetch=3,
        grid=(B, D // _CB),
        in_specs=[
            pl.BlockSpec((1, S, _CB), lambda b, c, *_: (b, 0, c)),
        ],
        out_specs=pl.BlockSpec((1, 1, 1, _CB), lambda b, c, *_: (b, 0, 0, c)),
        scratch_shapes=[pltpu.VMEM((S, _CB), jnp.int32)],
    )
    out = pl.pallas_call(
        _trim_kernel,
        grid_spec=grid_spec,
        out_shape=jax.ShapeDtypeStruct((B, 1, 1, D), jnp.float32),
        compiler_params=pltpu.CompilerParams(
            dimension_semantics=("parallel", "parallel"),
        ),
    )(n, k_lo, k_hi, outputs)
    return out
